# baseline (speedup 1.0000x reference)
import functools

import jax
import jax.numpy as jnp
from jax import lax
from jax.experimental import pallas as pl
from jax.experimental.pallas import tpu as pltpu
from jax.experimental.pallas import tpu_sc as plsc

_EPS = 1e-5


def _gate_kernel(x_ref, wg_ref, bg_ref, e0_ref, e1_ref, w0_ref, w1_ref,
                 xb_ref, *, n_experts):
    x = x_ref[...]
    logits = jnp.dot(x, wg_ref[...], preferred_element_type=jnp.float32)
    logits = logits + bg_ref[...]
    eids = jax.lax.broadcasted_iota(jnp.int32, logits.shape, 1)
    v0 = jnp.max(logits, axis=1, keepdims=True)
    e0 = jnp.min(jnp.where(logits == v0, eids, n_experts), axis=1, keepdims=True)
    masked = jnp.where(eids == e0, -jnp.inf, logits)
    v1 = jnp.max(masked, axis=1, keepdims=True)
    e1 = jnp.min(jnp.where(masked == v1, eids, n_experts), axis=1, keepdims=True)
    w0 = 1.0 / (1.0 + jnp.exp(v1 - v0))
    e0_ref[...] = e0
    e1_ref[...] = e1
    w0_ref[...] = w0
    w1_ref[...] = 1.0 - w0
    xb_ref[...] = x.astype(jnp.bfloat16)


def _gate(xf, Wg, bg, n, h, e):
    rb = 512
    return pl.pallas_call(
        functools.partial(_gate_kernel, n_experts=e),
        grid=(n // rb,),
        in_specs=[
            pl.BlockSpec((rb, h), lambda i: (i, 0)),
            pl.BlockSpec((h, e), lambda i: (0, 0)),
            pl.BlockSpec((1, e), lambda i: (0, 0)),
        ],
        out_specs=[
            pl.BlockSpec((rb, 1), lambda i: (i, 0)),
            pl.BlockSpec((rb, 1), lambda i: (i, 0)),
            pl.BlockSpec((rb, 1), lambda i: (i, 0)),
            pl.BlockSpec((rb, 1), lambda i: (i, 0)),
            pl.BlockSpec((rb, h), lambda i: (i, 0)),
        ],
        out_shape=[
            jax.ShapeDtypeStruct((n, 1), jnp.int32),
            jax.ShapeDtypeStruct((n, 1), jnp.int32),
            jax.ShapeDtypeStruct((n, 1), jnp.float32),
            jax.ShapeDtypeStruct((n, 1), jnp.float32),
            jax.ShapeDtypeStruct((n, h), jnp.bfloat16),
        ],
    )(xf, Wg, bg.reshape(1, e))


def _sc_gather_rows(table, idx):
    t, d = table.shape
    p = idx.shape[0]
    nw = 32
    rows_w = p // nw
    ch = 32
    nch = rows_w // ch
    mesh = plsc.VectorSubcoreMesh(core_axis_name="c", subcore_axis_name="s")

    @functools.partial(
        pl.kernel,
        mesh=mesh,
        out_type=jax.ShapeDtypeStruct((p, d), jnp.float32),
        scratch_types=[
            pltpu.VMEM((rows_w,), jnp.int32),
            pltpu.VMEM((ch, d), jnp.float32),
            pltpu.VMEM((ch, d), jnp.float32),
            pltpu.SemaphoreType.DMA,
            pltpu.SemaphoreType.DMA,
        ],
    )
    def k(table_hbm, idx_hbm, out_hbm, idx_v, buf0, buf1, sem0, sem1):
        wid = lax.axis_index("s") * 2 + lax.axis_index("c")
        base = wid * rows_w
        pltpu.sync_copy(idx_hbm.at[pl.ds(base, rows_w)], idx_v)
        bufs = (buf0, buf1)
        sems = (sem0, sem1)
        pltpu.async_copy(table_hbm.at[idx_v.at[pl.ds(0, ch)]], bufs[0], sems[0])
        for c in range(nch):
            cur, sem = bufs[c % 2], sems[c % 2]
            if c + 1 < nch:
                pltpu.async_copy(
                    table_hbm.at[idx_v.at[pl.ds((c + 1) * ch, ch)]],
                    bufs[(c + 1) % 2], sems[(c + 1) % 2])
            pltpu.make_async_copy(
                table_hbm.at[idx_v.at[pl.ds(c * ch, ch)]], cur, sem).wait()
            pltpu.sync_copy(cur, out_hbm.at[pl.ds(base + c * ch, ch)])

    return k(table, idx)


def _gather_bf16_rows(table_bf16, idx):
    n, d = table_bf16.shape
    view = jax.lax.bitcast_convert_type(
        table_bf16.reshape(n, d // 2, 2), jnp.float32)
    out = _sc_gather_rows(view, idx)
    out = jax.lax.bitcast_convert_type(out, jnp.bfloat16)
    return out.reshape(idx.shape[0], d)


def _gmm_kernel(sb_ref, se_ref, gs_ref, ge_ref,
                xs_ref, w1_ref, b1_ref, w2_ref, b2_ref, out_ref, *, r):
    g = pl.program_id(0)
    b = sb_ref[g]
    rows = b * r + jax.lax.broadcasted_iota(jnp.int32, (r, 1), 0)
    mask = (rows >= gs_ref[g]) & (rows < ge_ref[g])

    a = xs_ref[...]
    h = jnp.dot(a, w1_ref[0], preferred_element_type=jnp.float32) + b1_ref[0]
    h = jnp.maximum(h, 0.0).astype(jnp.bfloat16)
    y = jnp.dot(h, w2_ref[0], preferred_element_type=jnp.float32) + b2_ref[0]
    yb = y.astype(jnp.bfloat16)

    first = sb_ref[jnp.maximum(g - 1, 0)] != b
    first = jnp.logical_or(g == 0, first)

    @pl.when(first)
    def _():
        out_ref[...] = jnp.where(mask, yb, jnp.bfloat16(0.0))

    @pl.when(jnp.logical_not(first))
    def _():
        out_ref[...] = jnp.where(mask, yb, out_ref[...])


def _grouped_ffn(xs, w1b, b1, w2b, b2, sb, se, gs, ge, n_steps, r, h, f, e):
    p = xs.shape[0]
    grid_spec = pltpu.PrefetchScalarGridSpec(
        num_scalar_prefetch=4,
        grid=(n_steps,),
        in_specs=[
            pl.BlockSpec((r, h), lambda g, sb, se, gs, ge: (sb[g], 0)),
            pl.BlockSpec((1, h, f), lambda g, sb, se, gs, ge: (se[g], 0, 0)),
            pl.BlockSpec((1, 1, f), lambda g, sb, se, gs, ge: (se[g], 0, 0)),
            pl.BlockSpec((1, f, h), lambda g, sb, se, gs, ge: (se[g], 0, 0)),
            pl.BlockSpec((1, 1, h), lambda g, sb, se, gs, ge: (se[g], 0, 0)),
        ],
        out_specs=pl.BlockSpec((r, h), lambda g, sb, se, gs, ge: (sb[g], 0)),
    )
    return pl.pallas_call(
        functools.partial(_gmm_kernel, r=r),
        grid_spec=grid_spec,
        out_shape=jax.ShapeDtypeStruct((p, h), jnp.bfloat16),
        compiler_params=pltpu.CompilerParams(
            dimension_semantics=("arbitrary",),
        ),
    )(sb, se, gs, ge, xs, w1b, b1.reshape(e, 1, f), w2b, b2.reshape(e, 1, h))


def _combine_ln_kernel(x_ref, y0_ref, y1_ref, w0_ref, w1_ref,
                       gamma_ref, beta_ref, out_ref):
    y0 = y0_ref[...].astype(jnp.float32)
    y1 = y1_ref[...].astype(jnp.float32)
    z = x_ref[...] + w0_ref[...] * y0 + w1_ref[...] * y1
    mean = jnp.mean(z, axis=1, keepdims=True)
    zc = z - mean
    var = jnp.mean(zc * zc, axis=1, keepdims=True)
    out_ref[...] = zc * jax.lax.rsqrt(var + _EPS) * gamma_ref[...] + beta_ref[...]


def _combine_ln(xf, yg, w0, w1, gamma, beta, n, h):
    rb = 512
    nb = n // rb
    return pl.pallas_call(
        _combine_ln_kernel,
        grid=(nb,),
        in_specs=[
            pl.BlockSpec((rb, h), lambda i: (i, 0)),
            pl.BlockSpec((rb, h), lambda i: (i, 0)),
            pl.BlockSpec((rb, h), lambda i: (i + nb, 0)),
            pl.BlockSpec((rb, 1), lambda i: (i, 0)),
            pl.BlockSpec((rb, 1), lambda i: (i, 0)),
            pl.BlockSpec((1, h), lambda i: (0, 0)),
            pl.BlockSpec((1, h), lambda i: (0, 0)),
        ],
        out_specs=pl.BlockSpec((rb, h), lambda i: (i, 0)),
        out_shape=jax.ShapeDtypeStruct((n, h), jnp.float32),
    )(xf, yg, yg, w0, w1, gamma.reshape(1, h), beta.reshape(1, h))


def kernel(x, Wg, bg, W1, b1, W2, b2, gamma, beta):
    B, S, H = x.shape
    E = Wg.shape[1]
    F = W1.shape[2]
    N = B * S
    P = 2 * N
    R = 256
    M = P // R
    G = M + E - 1

    xf = x.reshape(N, H)
    w1b = W1.astype(jnp.bfloat16)
    w2b = W2.astype(jnp.bfloat16)

    e0c, e1c, w0c, w1c, xbf = _gate(xf, Wg, bg, N, H, E)
    e0 = e0c[:, 0]
    e1 = e1c[:, 0]

    pe = jnp.concatenate([e0, e1])
    onehot = (pe[:, None] == jnp.arange(E)[None, :])
    counts = jnp.sum(onehot, axis=0, dtype=jnp.int32)
    ends = jnp.cumsum(counts)
    starts = ends - counts
    rank = jnp.cumsum(onehot.astype(jnp.int32), axis=0) - onehot.astype(jnp.int32)
    pos = starts[pe] + jnp.sum(jnp.where(onehot, rank, 0), axis=1)
    arange_p = jnp.arange(P, dtype=jnp.int32)
    sort_idx = jnp.zeros((P,), jnp.int32).at[pos].set(arange_p)
    st = (sort_idx % N).astype(jnp.int32)
    inv = pos.astype(jnp.int32)

    bidx = jnp.arange(M, dtype=jnp.int32)
    present = ((starts[None, :] < (bidx[:, None] + 1) * R)
               & (ends[None, :] > bidx[:, None] * R))
    flat = jnp.nonzero(present.ravel(), size=G, fill_value=M * E - 1)[0]
    flat = flat.astype(jnp.int32)
    sb = flat // E
    se = flat % E
    gs = starts[se].astype(jnp.int32)
    ge = ends[se].astype(jnp.int32)

    xs = _gather_bf16_rows(xbf, st)
    ys = _grouped_ffn(xs, w1b, b1, w2b, b2, sb, se, gs, ge, G, R, H, F, E)
    yg = _gather_bf16_rows(ys, inv)
    out = _combine_ln(xf, yg, w0c, w1c, gamma, beta, N, H)
    return out.reshape(B, S, H)


# speedup vs baseline: 2.0388x; 2.0388x over previous
import functools

import jax
import jax.numpy as jnp
from jax import lax
from jax.experimental import pallas as pl
from jax.experimental.pallas import tpu as pltpu
from jax.experimental.pallas import tpu_sc as plsc

_EPS = 1e-5


def _pack_bf16_halves(xb):
    r, c2 = xb.shape
    c = c2 // 2
    ev = jax.lax.bitcast_convert_type(xb[:, :c], jnp.uint16).astype(jnp.uint32)
    od = jax.lax.bitcast_convert_type(xb[:, c:], jnp.uint16).astype(jnp.uint32)
    return jax.lax.bitcast_convert_type(ev | (od << 16), jnp.float32)


def _unpack_bf16_halves(xp):
    u = jax.lax.bitcast_convert_type(xp, jnp.uint32)
    ev = jax.lax.bitcast_convert_type((u & 0xFFFF).astype(jnp.uint16),
                                      jnp.bfloat16)
    od = jax.lax.bitcast_convert_type((u >> 16).astype(jnp.uint16),
                                      jnp.bfloat16)
    return jnp.concatenate([ev, od], axis=1)


def _gate_kernel(x_ref, wg_ref, bg_ref, e0_ref, e1_ref, w0_ref, w1_ref,
                 xb_ref, *, n_experts):
    x = x_ref[...]
    logits = jnp.dot(x, wg_ref[...], preferred_element_type=jnp.float32)
    logits = logits + bg_ref[...]
    eids = jax.lax.broadcasted_iota(jnp.int32, logits.shape, 1)
    v0 = jnp.max(logits, axis=1, keepdims=True)
    e0 = jnp.min(jnp.where(logits == v0, eids, n_experts), axis=1, keepdims=True)
    masked = jnp.where(eids == e0, -jnp.inf, logits)
    v1 = jnp.max(masked, axis=1, keepdims=True)
    e1 = jnp.min(jnp.where(masked == v1, eids, n_experts), axis=1, keepdims=True)
    w0 = 1.0 / (1.0 + jnp.exp(v1 - v0))
    e0_ref[...] = e0
    e1_ref[...] = e1
    w0_ref[...] = w0
    w1_ref[...] = 1.0 - w0
    xb_ref[...] = _pack_bf16_halves(x.astype(jnp.bfloat16))


def _gate(xf, Wg, bg, n, h, e):
    rb = 512
    return pl.pallas_call(
        functools.partial(_gate_kernel, n_experts=e),
        grid=(n // rb,),
        in_specs=[
            pl.BlockSpec((rb, h), lambda i: (i, 0)),
            pl.BlockSpec((h, e), lambda i: (0, 0)),
            pl.BlockSpec((1, e), lambda i: (0, 0)),
        ],
        out_specs=[
            pl.BlockSpec((rb, 1), lambda i: (i, 0)),
            pl.BlockSpec((rb, 1), lambda i: (i, 0)),
            pl.BlockSpec((rb, 1), lambda i: (i, 0)),
            pl.BlockSpec((rb, 1), lambda i: (i, 0)),
            pl.BlockSpec((rb, h // 2), lambda i: (i, 0)),
        ],
        out_shape=[
            jax.ShapeDtypeStruct((n, 1), jnp.int32),
            jax.ShapeDtypeStruct((n, 1), jnp.int32),
            jax.ShapeDtypeStruct((n, 1), jnp.float32),
            jax.ShapeDtypeStruct((n, 1), jnp.float32),
            jax.ShapeDtypeStruct((n, h // 2), jnp.float32),
        ],
    )(xf, Wg, bg.reshape(1, e))


def _sc_gather_rows(table, idx):
    t, d = table.shape
    dt = table.dtype
    p = idx.shape[0]
    nw = 32
    rows_w = p // nw
    ch = 32
    nch = rows_w // ch
    mesh = plsc.VectorSubcoreMesh(core_axis_name="c", subcore_axis_name="s")

    @functools.partial(
        pl.kernel,
        mesh=mesh,
        out_type=jax.ShapeDtypeStruct((p, d), dt),
        scratch_types=[
            pltpu.VMEM((rows_w,), jnp.int32),
            pltpu.VMEM((ch, d), dt),
            pltpu.VMEM((ch, d), dt),
            pltpu.SemaphoreType.DMA,
            pltpu.SemaphoreType.DMA,
        ],
    )
    def k(table_hbm, idx_hbm, out_hbm, idx_v, buf0, buf1, sem0, sem1):
        wid = lax.axis_index("s") * 2 + lax.axis_index("c")
        base = wid * rows_w
        pltpu.sync_copy(idx_hbm.at[pl.ds(base, rows_w)], idx_v)
        bufs = (buf0, buf1)
        sems = (sem0, sem1)
        pltpu.async_copy(table_hbm.at[idx_v.at[pl.ds(0, ch)]], bufs[0], sems[0])
        for c in range(nch):
            cur, sem = bufs[c % 2], sems[c % 2]
            if c + 1 < nch:
                pltpu.async_copy(
                    table_hbm.at[idx_v.at[pl.ds((c + 1) * ch, ch)]],
                    bufs[(c + 1) % 2], sems[(c + 1) % 2])
            pltpu.make_async_copy(
                table_hbm.at[idx_v.at[pl.ds(c * ch, ch)]], cur, sem).wait()
            pltpu.sync_copy(cur, out_hbm.at[pl.ds(base + c * ch, ch)])

    return k(table, idx)


def _gmm_kernel(sb_ref, se_ref, gs_ref, ge_ref,
                xs_ref, w1_ref, b1_ref, w2_ref, b2_ref, out_ref, *, r):
    g = pl.program_id(0)
    b = sb_ref[g]
    rows = b * r + jax.lax.broadcasted_iota(jnp.int32, (r, 1), 0)
    mask = (rows >= gs_ref[g]) & (rows < ge_ref[g])

    a = _unpack_bf16_halves(xs_ref[...])
    h = jnp.dot(a, w1_ref[0], preferred_element_type=jnp.float32) + b1_ref[0]
    h = jnp.maximum(h, 0.0).astype(jnp.bfloat16)
    y = jnp.dot(h, w2_ref[0], preferred_element_type=jnp.float32) + b2_ref[0]
    yp = _pack_bf16_halves(y.astype(jnp.bfloat16))

    first = sb_ref[jnp.maximum(g - 1, 0)] != b
    first = jnp.logical_or(g == 0, first)

    @pl.when(first)
    def _():
        out_ref[...] = jnp.where(mask, yp, 0.0)

    @pl.when(jnp.logical_not(first))
    def _():
        out_ref[...] = jnp.where(mask, yp, out_ref[...])


def _grouped_ffn(xs, w1b, b1, w2b, b2, sb, se, gs, ge, n_steps, r, h, f, e):
    p = xs.shape[0]
    grid_spec = pltpu.PrefetchScalarGridSpec(
        num_scalar_prefetch=4,
        grid=(n_steps,),
        in_specs=[
            pl.BlockSpec((r, h // 2), lambda g, sb, se, gs, ge: (sb[g], 0)),
            pl.BlockSpec((1, h, f), lambda g, sb, se, gs, ge: (se[g], 0, 0)),
            pl.BlockSpec((1, 1, f), lambda g, sb, se, gs, ge: (se[g], 0, 0)),
            pl.BlockSpec((1, f, h), lambda g, sb, se, gs, ge: (se[g], 0, 0)),
            pl.BlockSpec((1, 1, h), lambda g, sb, se, gs, ge: (se[g], 0, 0)),
        ],
        out_specs=pl.BlockSpec((r, h // 2), lambda g, sb, se, gs, ge: (sb[g], 0)),
    )
    return pl.pallas_call(
        functools.partial(_gmm_kernel, r=r),
        grid_spec=grid_spec,
        out_shape=jax.ShapeDtypeStruct((p, h // 2), jnp.float32),
        compiler_params=pltpu.CompilerParams(
            dimension_semantics=("arbitrary",),
        ),
    )(sb, se, gs, ge, xs, w1b, b1.reshape(e, 1, f), w2b, b2.reshape(e, 1, h))


def _combine_ln_kernel(x_ref, y0_ref, y1_ref, w0_ref, w1_ref,
                       gamma_ref, beta_ref, out_ref):
    y0 = _unpack_bf16_halves(y0_ref[...]).astype(jnp.float32)
    y1 = _unpack_bf16_halves(y1_ref[...]).astype(jnp.float32)
    z = x_ref[...] + w0_ref[...] * y0 + w1_ref[...] * y1
    mean = jnp.mean(z, axis=1, keepdims=True)
    zc = z - mean
    var = jnp.mean(zc * zc, axis=1, keepdims=True)
    out_ref[...] = zc * jax.lax.rsqrt(var + _EPS) * gamma_ref[...] + beta_ref[...]


def _combine_ln(xf, yg, w0, w1, gamma, beta, n, h):
    rb = 512
    nb = n // rb
    return pl.pallas_call(
        _combine_ln_kernel,
        grid=(nb,),
        in_specs=[
            pl.BlockSpec((rb, h), lambda i: (i, 0)),
            pl.BlockSpec((rb, h // 2), lambda i: (i, 0)),
            pl.BlockSpec((rb, h // 2), lambda i: (i + nb, 0)),
            pl.BlockSpec((rb, 1), lambda i: (i, 0)),
            pl.BlockSpec((rb, 1), lambda i: (i, 0)),
            pl.BlockSpec((1, h), lambda i: (0, 0)),
            pl.BlockSpec((1, h), lambda i: (0, 0)),
        ],
        out_specs=pl.BlockSpec((rb, h), lambda i: (i, 0)),
        out_shape=jax.ShapeDtypeStruct((n, h), jnp.float32),
    )(xf, yg, yg, w0, w1, gamma.reshape(1, h), beta.reshape(1, h))


def kernel(x, Wg, bg, W1, b1, W2, b2, gamma, beta):
    B, S, H = x.shape
    E = Wg.shape[1]
    F = W1.shape[2]
    N = B * S
    P = 2 * N
    R = 256
    M = P // R
    G = M + E - 1

    xf = x.reshape(N, H)
    w1b = W1.astype(jnp.bfloat16)
    w2b = W2.astype(jnp.bfloat16)

    e0c, e1c, w0c, w1c, xbf = _gate(xf, Wg, bg, N, H, E)
    e0 = e0c[:, 0]
    e1 = e1c[:, 0]

    pe = jnp.concatenate([e0, e1])
    onehot = (pe[:, None] == jnp.arange(E)[None, :])
    counts = jnp.sum(onehot, axis=0, dtype=jnp.int32)
    ends = jnp.cumsum(counts)
    starts = ends - counts
    rank = jnp.cumsum(onehot.astype(jnp.int32), axis=0) - onehot.astype(jnp.int32)
    pos = starts[pe] + jnp.sum(jnp.where(onehot, rank, 0), axis=1)
    arange_p = jnp.arange(P, dtype=jnp.int32)
    sort_idx = jnp.zeros((P,), jnp.int32).at[pos].set(arange_p)
    st = (sort_idx % N).astype(jnp.int32)
    inv = pos.astype(jnp.int32)

    bidx = jnp.arange(M, dtype=jnp.int32)
    present = ((starts[None, :] < (bidx[:, None] + 1) * R)
               & (ends[None, :] > bidx[:, None] * R))
    flat = jnp.nonzero(present.ravel(), size=G, fill_value=M * E - 1)[0]
    flat = flat.astype(jnp.int32)
    sb = flat // E
    se = flat % E
    gs = starts[se].astype(jnp.int32)
    ge = ends[se].astype(jnp.int32)

    xs = _sc_gather_rows(xbf, st)
    ys = _grouped_ffn(xs, w1b, b1, w2b, b2, sb, se, gs, ge, G, R, H, F, E)
    yg = _sc_gather_rows(ys, inv)
    out = _combine_ln(xf, yg, w0c, w1c, gamma, beta, N, H)
    return out.reshape(B, S, H)


# speedup vs baseline: 2.0924x; 1.0263x over previous
import functools

import jax
import jax.numpy as jnp
from jax import lax
from jax.experimental import pallas as pl
from jax.experimental.pallas import tpu as pltpu
from jax.experimental.pallas import tpu_sc as plsc

_EPS = 1e-5


def _pack_bf16_halves(xb):
    r, c2 = xb.shape
    c = c2 // 2
    ev = jax.lax.bitcast_convert_type(xb[:, :c], jnp.uint16).astype(jnp.uint32)
    od = jax.lax.bitcast_convert_type(xb[:, c:], jnp.uint16).astype(jnp.uint32)
    return jax.lax.bitcast_convert_type(ev | (od << 16), jnp.float32)


def _unpack_bf16_halves(xp):
    u = jax.lax.bitcast_convert_type(xp, jnp.uint32)
    ev = jax.lax.bitcast_convert_type((u & 0xFFFF).astype(jnp.uint16),
                                      jnp.bfloat16)
    od = jax.lax.bitcast_convert_type((u >> 16).astype(jnp.uint16),
                                      jnp.bfloat16)
    return jnp.concatenate([ev, od], axis=1)


def _gate_kernel(x_ref, wg_ref, bg_ref, e0_ref, e1_ref, w0_ref, w1_ref,
                 xb_ref, *, n_experts):
    x = x_ref[...]
    logits = jnp.dot(x, wg_ref[...], preferred_element_type=jnp.float32)
    logits = logits + bg_ref[...]
    eids = jax.lax.broadcasted_iota(jnp.int32, logits.shape, 1)
    v0 = jnp.max(logits, axis=1, keepdims=True)
    e0 = jnp.min(jnp.where(logits == v0, eids, n_experts), axis=1, keepdims=True)
    masked = jnp.where(eids == e0, -jnp.inf, logits)
    v1 = jnp.max(masked, axis=1, keepdims=True)
    e1 = jnp.min(jnp.where(masked == v1, eids, n_experts), axis=1, keepdims=True)
    w0 = 1.0 / (1.0 + jnp.exp(v1 - v0))
    e0_ref[...] = e0
    e1_ref[...] = e1
    w0_ref[...] = w0
    w1_ref[...] = 1.0 - w0
    xb_ref[...] = _pack_bf16_halves(x.astype(jnp.bfloat16))


def _gate(xf, Wg, bg, n, h, e):
    rb = 1024
    return pl.pallas_call(
        functools.partial(_gate_kernel, n_experts=e),
        grid=(n // rb,),
        in_specs=[
            pl.BlockSpec((rb, h), lambda i: (i, 0)),
            pl.BlockSpec((h, e), lambda i: (0, 0)),
            pl.BlockSpec((1, e), lambda i: (0, 0)),
        ],
        out_specs=[
            pl.BlockSpec((rb, 1), lambda i: (i, 0)),
            pl.BlockSpec((rb, 1), lambda i: (i, 0)),
            pl.BlockSpec((rb, 1), lambda i: (i, 0)),
            pl.BlockSpec((rb, 1), lambda i: (i, 0)),
            pl.BlockSpec((rb, h // 2), lambda i: (i, 0)),
        ],
        out_shape=[
            jax.ShapeDtypeStruct((n, 1), jnp.int32),
            jax.ShapeDtypeStruct((n, 1), jnp.int32),
            jax.ShapeDtypeStruct((n, 1), jnp.float32),
            jax.ShapeDtypeStruct((n, 1), jnp.float32),
            jax.ShapeDtypeStruct((n, h // 2), jnp.float32),
        ],
    )(xf, Wg, bg.reshape(1, e))


def _sc_gather_rows(table, idx):
    t, d = table.shape
    dt = table.dtype
    p = idx.shape[0]
    nw = 32
    rows_w = p // nw
    ch = 64
    nch = rows_w // ch
    mesh = plsc.VectorSubcoreMesh(core_axis_name="c", subcore_axis_name="s")

    @functools.partial(
        pl.kernel,
        mesh=mesh,
        out_type=jax.ShapeDtypeStruct((p, d), dt),
        scratch_types=[
            pltpu.VMEM((rows_w,), jnp.int32),
            pltpu.VMEM((ch, d), dt),
            pltpu.VMEM((ch, d), dt),
            pltpu.SemaphoreType.DMA,
            pltpu.SemaphoreType.DMA,
        ],
    )
    def k(table_hbm, idx_hbm, out_hbm, idx_v, buf0, buf1, sem0, sem1):
        wid = lax.axis_index("s") * 2 + lax.axis_index("c")
        base = wid * rows_w
        pltpu.sync_copy(idx_hbm.at[pl.ds(base, rows_w)], idx_v)
        bufs = (buf0, buf1)
        sems = (sem0, sem1)
        pltpu.async_copy(table_hbm.at[idx_v.at[pl.ds(0, ch)]], bufs[0], sems[0])
        for c in range(nch):
            cur, sem = bufs[c % 2], sems[c % 2]
            if c + 1 < nch:
                pltpu.async_copy(
                    table_hbm.at[idx_v.at[pl.ds((c + 1) * ch, ch)]],
                    bufs[(c + 1) % 2], sems[(c + 1) % 2])
            pltpu.make_async_copy(
                table_hbm.at[idx_v.at[pl.ds(c * ch, ch)]], cur, sem).wait()
            pltpu.sync_copy(cur, out_hbm.at[pl.ds(base + c * ch, ch)])

    return k(table, idx)


def _gmm_kernel(sb_ref, se_ref, gs_ref, ge_ref,
                xs_ref, w1_ref, b1_ref, w2_ref, b2_ref, out_ref, *, r):
    g = pl.program_id(0)
    b = sb_ref[g]
    rows = b * r + jax.lax.broadcasted_iota(jnp.int32, (r, 1), 0)
    mask = (rows >= gs_ref[g]) & (rows < ge_ref[g])

    a = _unpack_bf16_halves(xs_ref[...])
    h = jnp.dot(a, w1_ref[0], preferred_element_type=jnp.float32) + b1_ref[0]
    h = jnp.maximum(h, 0.0).astype(jnp.bfloat16)
    y = jnp.dot(h, w2_ref[0], preferred_element_type=jnp.float32) + b2_ref[0]
    yp = _pack_bf16_halves(y.astype(jnp.bfloat16))

    first = sb_ref[jnp.maximum(g - 1, 0)] != b
    first = jnp.logical_or(g == 0, first)

    @pl.when(first)
    def _():
        out_ref[...] = jnp.where(mask, yp, 0.0)

    @pl.when(jnp.logical_not(first))
    def _():
        out_ref[...] = jnp.where(mask, yp, out_ref[...])


def _grouped_ffn(xs, w1b, b1, w2b, b2, sb, se, gs, ge, n_steps, r, h, f, e):
    p = xs.shape[0]
    grid_spec = pltpu.PrefetchScalarGridSpec(
        num_scalar_prefetch=4,
        grid=(n_steps,),
        in_specs=[
            pl.BlockSpec((r, h // 2), lambda g, sb, se, gs, ge: (sb[g], 0)),
            pl.BlockSpec((1, h, f), lambda g, sb, se, gs, ge: (se[g], 0, 0)),
            pl.BlockSpec((1, 1, f), lambda g, sb, se, gs, ge: (se[g], 0, 0)),
            pl.BlockSpec((1, f, h), lambda g, sb, se, gs, ge: (se[g], 0, 0)),
            pl.BlockSpec((1, 1, h), lambda g, sb, se, gs, ge: (se[g], 0, 0)),
        ],
        out_specs=pl.BlockSpec((r, h // 2), lambda g, sb, se, gs, ge: (sb[g], 0)),
    )
    return pl.pallas_call(
        functools.partial(_gmm_kernel, r=r),
        grid_spec=grid_spec,
        out_shape=jax.ShapeDtypeStruct((p, h // 2), jnp.float32),
        compiler_params=pltpu.CompilerParams(
            dimension_semantics=("arbitrary",),
        ),
    )(sb, se, gs, ge, xs, w1b, b1.reshape(e, 1, f), w2b, b2.reshape(e, 1, h))


def _combine_ln_kernel(x_ref, y0_ref, y1_ref, w0_ref, w1_ref,
                       gamma_ref, beta_ref, out_ref):
    y0 = _unpack_bf16_halves(y0_ref[...]).astype(jnp.float32)
    y1 = _unpack_bf16_halves(y1_ref[...]).astype(jnp.float32)
    z = x_ref[...] + w0_ref[...] * y0 + w1_ref[...] * y1
    mean = jnp.mean(z, axis=1, keepdims=True)
    zc = z - mean
    var = jnp.mean(zc * zc, axis=1, keepdims=True)
    out_ref[...] = zc * jax.lax.rsqrt(var + _EPS) * gamma_ref[...] + beta_ref[...]


def _combine_ln(xf, yg, w0, w1, gamma, beta, n, h):
    rb = 1024
    nb = n // rb
    return pl.pallas_call(
        _combine_ln_kernel,
        grid=(nb,),
        in_specs=[
            pl.BlockSpec((rb, h), lambda i: (i, 0)),
            pl.BlockSpec((rb, h // 2), lambda i: (i, 0)),
            pl.BlockSpec((rb, h // 2), lambda i: (i + nb, 0)),
            pl.BlockSpec((rb, 1), lambda i: (i, 0)),
            pl.BlockSpec((rb, 1), lambda i: (i, 0)),
            pl.BlockSpec((1, h), lambda i: (0, 0)),
            pl.BlockSpec((1, h), lambda i: (0, 0)),
        ],
        out_specs=pl.BlockSpec((rb, h), lambda i: (i, 0)),
        out_shape=jax.ShapeDtypeStruct((n, h), jnp.float32),
    )(xf, yg, yg, w0, w1, gamma.reshape(1, h), beta.reshape(1, h))


def kernel(x, Wg, bg, W1, b1, W2, b2, gamma, beta):
    B, S, H = x.shape
    E = Wg.shape[1]
    F = W1.shape[2]
    N = B * S
    P = 2 * N
    R = 256
    M = P // R
    G = M + E - 1

    xf = x.reshape(N, H)
    w1b = W1.astype(jnp.bfloat16)
    w2b = W2.astype(jnp.bfloat16)

    e0c, e1c, w0c, w1c, xbf = _gate(xf, Wg, bg, N, H, E)
    e0 = e0c[:, 0]
    e1 = e1c[:, 0]

    pe = jnp.concatenate([e0, e1])
    onehot = (pe[:, None] == jnp.arange(E)[None, :])
    counts = jnp.sum(onehot, axis=0, dtype=jnp.int32)
    ends = jnp.cumsum(counts)
    starts = ends - counts
    rank = jnp.cumsum(onehot.astype(jnp.int32), axis=0) - onehot.astype(jnp.int32)
    pos = starts[pe] + jnp.sum(jnp.where(onehot, rank, 0), axis=1)
    arange_p = jnp.arange(P, dtype=jnp.int32)
    sort_idx = jnp.zeros((P,), jnp.int32).at[pos].set(arange_p)
    st = (sort_idx % N).astype(jnp.int32)
    inv = pos.astype(jnp.int32)

    bidx = jnp.arange(M, dtype=jnp.int32)
    present = ((starts[None, :] < (bidx[:, None] + 1) * R)
               & (ends[None, :] > bidx[:, None] * R))
    elo = jnp.sum((ends[None, :] <= bidx[:, None] * R).astype(jnp.int32), axis=1)
    nb = jnp.sum(present.astype(jnp.int32), axis=1)
    cum_nb = jnp.cumsum(nb)
    cum_x = cum_nb - nb
    gidx = jnp.arange(G, dtype=jnp.int32)
    sb = jnp.sum((gidx[:, None] >= cum_nb[None, :]).astype(jnp.int32), axis=1)
    sb = jnp.minimum(sb, M - 1).astype(jnp.int32)
    se = jnp.minimum(elo[sb] + (gidx - cum_x[sb]), E - 1).astype(jnp.int32)
    gs = starts[se].astype(jnp.int32)
    ge = ends[se].astype(jnp.int32)

    xs = _sc_gather_rows(xbf, st)
    ys = _grouped_ffn(xs, w1b, b1, w2b, b2, sb, se, gs, ge, G, R, H, F, E)
    yg = _sc_gather_rows(ys, inv)
    out = _combine_ln(xf, yg, w0c, w1c, gamma, beta, N, H)
    return out.reshape(B, S, H)
